# baseline (speedup 1.0000x reference)
import functools

import jax
import jax.numpy as jnp
from jax import lax
from jax.experimental import pallas as pl
from jax.experimental.pallas import tpu as pltpu
from jax.experimental.pallas import tpu_sc as plsc

_SLAB = 80
_NBUF = 3
_LANES = 16


@functools.lru_cache(maxsize=None)
def _build_gather(n, k, d):
    n_rows = n * k
    slab8 = 8 * k
    info = plsc.get_sparse_core_info()
    nc, ns = info.num_cores, info.num_subcores
    nw = nc * ns
    assert n_rows % nw == 0
    b_per_w = n_rows // nw
    assert b_per_w % _SLAB == 0 and _SLAB % _LANES == 0 and _SLAB <= 128
    assert slab8 & (slab8 - 1) == 0
    n_slabs = b_per_w // _SLAB
    assert n_slabs % _NBUF == 2
    assert n % ns == 0
    g_load = (b_per_w // slab8 + 2) * slab8

    mesh = plsc.VectorSubcoreMesh(core_axis_name="c", subcore_axis_name="s")

    @functools.partial(
        pl.kernel,
        mesh=mesh,
        out_type=jax.ShapeDtypeStruct((n_rows, d), jnp.float32),
        scratch_types=[
            pltpu.VMEM_SHARED((n, d), jnp.float32),
            pltpu.VMEM((g_load,), jnp.int32),
        ]
        + [pltpu.VMEM((_SLAB,), jnp.int32)] * _NBUF
        + [pltpu.VMEM((_SLAB, d), jnp.float32)] * _NBUF
        + [pltpu.SemaphoreType.DMA] * (2 * _NBUF),
        compiler_params=pltpu.CompilerParams(needs_layout_passes=False),
    )
    def gather_k(table_hbm, idx_hbm, out_hbm, tshared, gsrc, *rest):
        idxbs = rest[:_NBUF]
        bufs = rest[_NBUF:2 * _NBUF]
        gsem = rest[2 * _NBUF:3 * _NBUF]
        wsem = rest[3 * _NBUF:]
        sid = lax.axis_index("s")
        wid = sid * nc + lax.axis_index("c")
        p0 = wid * b_per_w

        rows_per_tile = (n // ns) & ~7
        rem = n - rows_per_tile * ns
        stripe = pltpu.make_async_copy(
            table_hbm.at[pl.ds(sid * rows_per_tile, rows_per_tile)],
            tshared.at[pl.ds(sid * rows_per_tile, rows_per_tile)],
            gsem[0],
        )
        stripe.start()
        if rem:
            @pl.when(sid == 0)
            def _():
                pltpu.sync_copy(
                    table_hbm.at[pl.ds(rows_per_tile * ns, rem)],
                    tshared.at[pl.ds(rows_per_tile * ns, rem)],
                )
        off = jnp.minimum((p0 // slab8) * slab8, n_rows - g_load)
        pltpu.sync_copy(idx_hbm.at[pl.ds(off, g_load)], gsrc)
        stripe.wait()
        plsc.subcore_barrier()

        def fill_idx(s, ph):
            for g in range(_SLAB // _LANES):
                p_vec = (p0 + s * _SLAB + g * _LANES) + lax.iota(jnp.int32, _LANES)
                t = p_vec & (slab8 - 1)
                src = (p_vec - t - off) + (t & 7) * k + (t >> 3)
                idxbs[ph][pl.ds(g * _LANES, _LANES)] = plsc.load_gather(gsrc, [src])

        def gather_desc(ph):
            return pltpu.make_async_copy(
                tshared.at[idxbs[ph]], bufs[ph], gsem[ph])

        def gather_desc_hbm(ph):
            return pltpu.make_async_copy(
                table_hbm.at[idxbs[ph]], bufs[ph], gsem[ph])

        def start_gather(s, ph):
            if isinstance(s, int):
                (gather_desc if s % 2 == 0 else gather_desc_hbm)(ph).start()
            else:
                @pl.when((s & 1) == 0)
                def _():
                    gather_desc(ph).start()

                @pl.when((s & 1) == 1)
                def _():
                    gather_desc_hbm(ph).start()

        def write_desc(s, ph):
            return pltpu.make_async_copy(
                bufs[ph],
                out_hbm.at[pl.ds(p0 + s * _SLAB, _SLAB)],
                wsem[ph],
            )

        def stage(s, ph, first=False, guard=True):
            if not first:
                def free_buf():
                    write_desc(s - _NBUF, ph).wait()
                if guard:
                    pl.when(s >= _NBUF)(free_buf)
                else:
                    free_buf()
            fill_idx(s, ph)
            start_gather(s, ph)
            if not first:
                gather_desc((ph - 1) % _NBUF).wait()
                write_desc(s - 1, (ph - 1) % _NBUF).start()

        stage(0, 0, first=True)

        def body(r, carry):
            s = r * _NBUF
            for i in range(1, _NBUF + 1):
                stage(s + i, i % _NBUF)
            return carry

        lax.fori_loop(0, (n_slabs - 2) // _NBUF, body, 0)

        last = n_slabs - 1
        stage(last, last % _NBUF, guard=False)
        gather_desc(last % _NBUF).wait()
        write_desc(last, last % _NBUF).start()
        for s in range(n_slabs - _NBUF + 1, n_slabs + 1):
            write_desc(s - 1, (s - 1) % _NBUF).wait()

    return gather_k


def kernel(logits, G):
    n, d = logits.shape
    k = G.shape[1]
    idx = G.astype(jnp.int32).reshape(-1)
    out = _build_gather(n, k, d)(logits, idx)
    return out.reshape(n // 8, k, 8, d).transpose(0, 2, 1, 3).reshape(n, k * d)


# speedup vs baseline: 1.3901x; 1.3901x over previous
import functools

import jax
import jax.numpy as jnp
from jax import lax
from jax.experimental import pallas as pl
from jax.experimental.pallas import tpu as pltpu
from jax.experimental.pallas import tpu_sc as plsc

_SLAB = 80
_NBUF = 3
_LANES = 16


@functools.lru_cache(maxsize=None)
def _build_gather(n, k, d):
    n_rows = n * k
    slab8 = 8 * k
    info = plsc.get_sparse_core_info()
    nc, ns = info.num_cores, info.num_subcores
    nw = nc * ns
    assert n_rows % nw == 0
    b_per_w = n_rows // nw
    assert b_per_w % _SLAB == 0 and _SLAB % _LANES == 0 and _SLAB <= 128
    assert slab8 & (slab8 - 1) == 0
    n_slabs = b_per_w // _SLAB
    assert n_slabs % _NBUF == 2
    assert n % ns == 0
    g_load = (b_per_w // slab8 + 2) * slab8

    mesh = plsc.VectorSubcoreMesh(core_axis_name="c", subcore_axis_name="s")

    @functools.partial(
        pl.kernel,
        mesh=mesh,
        out_type=jax.ShapeDtypeStruct((n_rows, d), jnp.float32),
        scratch_types=[
            pltpu.VMEM_SHARED((n, d), jnp.float32),
            pltpu.VMEM((g_load,), jnp.int32),
        ]
        + [pltpu.VMEM((_SLAB,), jnp.int32)] * _NBUF
        + [pltpu.VMEM((_SLAB, d), jnp.float32)] * _NBUF
        + [pltpu.SemaphoreType.DMA] * (2 * _NBUF),
        compiler_params=pltpu.CompilerParams(needs_layout_passes=False),
    )
    def gather_k(table_hbm, idx_hbm, out_hbm, tshared, gsrc, *rest):
        idxbs = rest[:_NBUF]
        bufs = rest[_NBUF:2 * _NBUF]
        gsem = rest[2 * _NBUF:3 * _NBUF]
        wsem = rest[3 * _NBUF:]
        sid = lax.axis_index("s")
        wid = sid * nc + lax.axis_index("c")
        p0 = wid * b_per_w

        rows_per_tile = (n // ns) & ~7
        rem = n - rows_per_tile * ns
        stripe = pltpu.make_async_copy(
            table_hbm.at[pl.ds(sid * rows_per_tile, rows_per_tile)],
            tshared.at[pl.ds(sid * rows_per_tile, rows_per_tile)],
            gsem[0],
        )
        stripe.start()
        if rem:
            @pl.when(sid == 0)
            def _():
                pltpu.sync_copy(
                    table_hbm.at[pl.ds(rows_per_tile * ns, rem)],
                    tshared.at[pl.ds(rows_per_tile * ns, rem)],
                )
        off = jnp.minimum((p0 // slab8) * slab8, n_rows - g_load)
        pltpu.sync_copy(idx_hbm.at[pl.ds(off, g_load)], gsrc)
        stripe.wait()
        plsc.subcore_barrier()

        def fill_idx(s, ph):
            for g in range(_SLAB // _LANES):
                p_vec = (p0 + s * _SLAB + g * _LANES) + lax.iota(jnp.int32, _LANES)
                t = p_vec & (slab8 - 1)
                src = (p_vec - t - off) + (t & 7) * k + (t >> 3)
                idxbs[ph][pl.ds(g * _LANES, _LANES)] = plsc.load_gather(gsrc, [src])

        def gather_desc(ph):
            return pltpu.make_async_copy(
                tshared.at[idxbs[ph]], bufs[ph], gsem[ph])

        def write_desc(s, ph):
            return pltpu.make_async_copy(
                bufs[ph],
                out_hbm.at[pl.ds(p0 + s * _SLAB, _SLAB)],
                wsem[ph],
            )

        def stage(s, ph, first=False, guard=True):
            if not first:
                def free_buf():
                    write_desc(s - _NBUF, ph).wait()
                if guard:
                    pl.when(s >= _NBUF)(free_buf)
                else:
                    free_buf()
            fill_idx(s, ph)
            gather_desc(ph).start()
            if not first:
                gather_desc((ph - 1) % _NBUF).wait()
                write_desc(s - 1, (ph - 1) % _NBUF).start()

        stage(0, 0, first=True)

        def body(r, carry):
            s = r * _NBUF
            for i in range(1, _NBUF + 1):
                stage(s + i, i % _NBUF)
            return carry

        lax.fori_loop(0, (n_slabs - 2) // _NBUF, body, 0)

        last = n_slabs - 1
        stage(last, last % _NBUF, guard=False)
        gather_desc(last % _NBUF).wait()
        write_desc(last, last % _NBUF).start()
        for s in range(n_slabs - _NBUF + 1, n_slabs + 1):
            write_desc(s - 1, (s - 1) % _NBUF).wait()

    return gather_k


def kernel(logits, G):
    n, d = logits.shape
    k = G.shape[1]
    idx = G.astype(jnp.int32).reshape(-1)
    out = _build_gather(n, k, d)(logits, idx)
    return out.reshape(n // 8, k, 8, d).transpose(0, 2, 1, 3).reshape(n, k * d)
